# baseline (speedup 1.0000x reference)
import jax
import jax.numpy as jnp
from jax.experimental import pallas as pl
from jax.experimental.pallas import tpu as pltpu

_K = 64
_BLK = 256


def _body(x_ref, we_ref, be_ref, wd_ref, bd_ref, sparse_ref, recon_ref,
          lat_s):
    lat_new = jnp.dot(x_ref[...], we_ref[...],
                      preferred_element_type=jnp.float32) + be_ref[...]

    lat = lat_s[...]

    ik = jax.lax.bitcast_convert_type(lat, jnp.int32)
    keys = ik ^ ((ik >> 31) & jnp.int32(0x7FFFFFFF))

    cnt0 = jnp.sum((keys >= 0).astype(jnp.int32), axis=1, keepdims=True)
    base = jnp.where(cnt0 >= _K, jnp.int32(0), jnp.int32(-(2**31)))
    for b in range(30, -1, -1):
        trial = base | jnp.int32(1 << b)
        cnt = jnp.sum((keys >= trial).astype(jnp.int32), axis=1, keepdims=True)
        base = jnp.where(cnt >= _K, trial, base)

    sparse = jnp.where(keys >= base, lat, jnp.float32(0.0))
    sparse_ref[...] = sparse
    recon_ref[...] = jnp.dot(sparse.astype(jnp.bfloat16), wd_ref[...],
                             preferred_element_type=jnp.float32) + bd_ref[...]

    lat_s[...] = lat_new


@jax.jit
def kernel(x, W_enc, b_enc, W_dec, b_dec):
    B, D_in = x.shape
    D_lat = W_enc.shape[0]
    nb = B // _BLK
    last = nb - 1
    sparse, recon = pl.pallas_call(
        _body,
        grid=(nb + 1,),
        in_specs=[
            pl.BlockSpec((_BLK, D_in), lambda i: (jnp.minimum(i, last), 0)),
            pl.BlockSpec((D_in, D_lat), lambda i: (0, 0)),
            pl.BlockSpec((1, D_lat), lambda i: (0, 0)),
            pl.BlockSpec((D_lat, D_in), lambda i: (0, 0)),
            pl.BlockSpec((1, D_in), lambda i: (0, 0)),
        ],
        out_specs=[
            pl.BlockSpec((_BLK, D_lat), lambda i: (jnp.maximum(i - 1, 0), 0)),
            pl.BlockSpec((_BLK, D_in), lambda i: (jnp.maximum(i - 1, 0), 0)),
        ],
        out_shape=[
            jax.ShapeDtypeStruct((B, D_lat), jnp.float32),
            jax.ShapeDtypeStruct((B, D_in), jnp.float32),
        ],
        scratch_shapes=[pltpu.VMEM((_BLK, D_lat), jnp.float32)],
    )(x.astype(jnp.bfloat16), W_enc.T.astype(jnp.bfloat16), b_enc[None, :],
      W_dec.T.astype(jnp.bfloat16), b_dec[None, :])
    return (recon, sparse)


# speedup vs baseline: 1.1280x; 1.1280x over previous
import jax
import jax.numpy as jnp
from jax.experimental import pallas as pl

_K = 64
_BLK = 256


def _count(cond):
    c = cond.astype(jnp.int16)
    w = c.shape[1]
    while w > 128:
        w //= 2
        c = c[:, :w] + c[:, w:]
    return jnp.sum(c.astype(jnp.int32), axis=1, keepdims=True)


def _search16(plane, target, bits=15, signed=True):
    if signed:
        cnt0 = _count(plane >= jnp.int16(0))
        base = jnp.where(cnt0 >= target, jnp.int32(0), jnp.int32(-(2**15)))
    else:
        base = jnp.zeros_like(target, dtype=jnp.int32)
    for b in range(bits - 1, -1, -1):
        trial = base | jnp.int32(1 << b)
        cnt = _count(plane >= trial.astype(jnp.int16))
        base = jnp.where(cnt >= target, trial, base)
    return base


def _body(x_ref, we_ref, be_ref, wd_ref, bd_ref, sparse_ref, recon_ref):
    lat = jnp.dot(x_ref[...], we_ref[...],
                  preferred_element_type=jnp.float32)
    lat = lat + be_ref[...]

    ik = jax.lax.bitcast_convert_type(lat, jnp.int32)
    keys = ik ^ ((ik >> 31) & jnp.int32(0x7FFFFFFF))
    hi = (keys >> 16).astype(jnp.int16)
    lo = ((keys & jnp.int32(0xFFFF)) - jnp.int32(32768)).astype(jnp.int16)

    h = _search16(hi, jnp.int32(_K))
    h16 = h.astype(jnp.int16)
    c1 = _count(hi > h16)
    need = jnp.int32(_K) - c1
    lom = jnp.where(hi == h16, lo, jnp.int16(-(2**15)))
    l = _search16(lom, need)

    thr = (h << 16) | (l + 32768)
    above = keys > thr
    eq = keys == thr
    need_eq = jnp.int32(_K) - jnp.sum(above.astype(jnp.int32), axis=1,
                                      keepdims=True)
    ncols = lat.shape[1]
    rev = jnp.int32(ncols - 1) - jax.lax.broadcasted_iota(jnp.int32, lat.shape, 1)
    ipm32 = eq.astype(jnp.int32) * (rev + jnp.int32(2**15)) - jnp.int32(2**15)
    ipm = ipm32.astype(jnp.int16)
    d = _search16(ipm, need_eq, bits=(ncols - 1).bit_length(), signed=False)
    mask = above | (eq & (rev >= d))
    sparse = jnp.where(mask, lat, jnp.float32(0.0))
    sparse_ref[...] = sparse
    recon_ref[...] = jnp.dot(sparse.astype(jnp.bfloat16), wd_ref[...],
                             preferred_element_type=jnp.float32) + bd_ref[...]


@jax.jit
def kernel(x, W_enc, b_enc, W_dec, b_dec):
    B, D_in = x.shape
    D_lat = W_enc.shape[0]
    grid = (B // _BLK,)
    sparse, recon = pl.pallas_call(
        _body,
        grid=grid,
        in_specs=[
            pl.BlockSpec((_BLK, D_in), lambda i: (i, 0)),
            pl.BlockSpec((D_in, D_lat), lambda i: (0, 0)),
            pl.BlockSpec((1, D_lat), lambda i: (0, 0)),
            pl.BlockSpec((D_lat, D_in), lambda i: (0, 0)),
            pl.BlockSpec((1, D_in), lambda i: (0, 0)),
        ],
        out_specs=[
            pl.BlockSpec((_BLK, D_lat), lambda i: (i, 0)),
            pl.BlockSpec((_BLK, D_in), lambda i: (i, 0)),
        ],
        out_shape=[
            jax.ShapeDtypeStruct((B, D_lat), jnp.float32),
            jax.ShapeDtypeStruct((B, D_in), jnp.float32),
        ],
    )(x.astype(jnp.bfloat16), W_enc.T.astype(jnp.bfloat16), b_enc[None, :],
      W_dec.T.astype(jnp.bfloat16), b_dec[None, :])
    return (recon, sparse)


# speedup vs baseline: 1.1809x; 1.0469x over previous
import jax
import jax.numpy as jnp
from jax.experimental import pallas as pl

_K = 64
_BLK = 256


def _count(cond):
    c = cond.astype(jnp.int16)
    w = c.shape[1]
    while w > 128:
        w //= 2
        c = c[:, :w] + c[:, w:]
    return jnp.sum(c.astype(jnp.int32), axis=1, keepdims=True)


def _search16(plane, target, bits=15, signed=True):
    if signed:
        cnt0 = _count(plane >= jnp.int16(0))
        base = jnp.where(cnt0 >= target, jnp.int32(0), jnp.int32(-(2**15)))
    else:
        base = jnp.zeros_like(target, dtype=jnp.int32)
    for b in range(bits - 1, -1, -1):
        trial = base | jnp.int32(1 << b)
        cnt = _count(plane >= trial.astype(jnp.int16))
        base = jnp.where(cnt >= target, trial, base)
    return base


_DN_T = (((1,), (1,)), ((), ()))


def _body(x_ref, we_ref, be_ref, wd_ref, bd_ref, sparse_ref, recon_ref):
    lat = jax.lax.dot_general(x_ref[...].astype(jnp.bfloat16), we_ref[...],
                              _DN_T, preferred_element_type=jnp.float32)
    lat = lat + be_ref[...]

    ik = jax.lax.bitcast_convert_type(lat, jnp.int32)
    keys = ik ^ ((ik >> 31) & jnp.int32(0x7FFFFFFF))
    hi = (keys >> 16).astype(jnp.int16)
    lo = ((keys & jnp.int32(0xFFFF)) - jnp.int32(32768)).astype(jnp.int16)

    h = _search16(hi, jnp.int32(_K))
    h16 = h.astype(jnp.int16)
    c1 = _count(hi > h16)
    need = jnp.int32(_K) - c1
    lom = jnp.where(hi == h16, lo, jnp.int16(-(2**15)))
    l = _search16(lom, need)

    thr = (h << 16) | (l + 32768)
    above = keys > thr
    eq = keys == thr
    need_eq = jnp.int32(_K) - jnp.sum(above.astype(jnp.int32), axis=1,
                                      keepdims=True)
    ncols = lat.shape[1]
    rev = jnp.int32(ncols - 1) - jax.lax.broadcasted_iota(jnp.int32, lat.shape, 1)
    ipm = jnp.where(eq, rev, jnp.int32(-(2**15))).astype(jnp.int16)
    d = _search16(ipm, need_eq, bits=(ncols - 1).bit_length(), signed=False)
    mask = above | (eq & (rev >= d))
    sparse = jnp.where(mask, lat, jnp.float32(0.0))
    sparse_ref[...] = sparse
    recon_ref[...] = jax.lax.dot_general(
        sparse.astype(jnp.bfloat16), wd_ref[...], _DN_T,
        preferred_element_type=jnp.float32) + bd_ref[...]


@jax.jit
def kernel(x, W_enc, b_enc, W_dec, b_dec):
    B, D_in = x.shape
    D_lat = W_enc.shape[0]
    grid = (B // _BLK,)
    sparse, recon = pl.pallas_call(
        _body,
        grid=grid,
        in_specs=[
            pl.BlockSpec((_BLK, D_in), lambda i: (i, 0)),
            pl.BlockSpec((D_lat, D_in), lambda i: (0, 0)),
            pl.BlockSpec((1, D_lat), lambda i: (0, 0)),
            pl.BlockSpec((D_in, D_lat), lambda i: (0, 0)),
            pl.BlockSpec((1, D_in), lambda i: (0, 0)),
        ],
        out_specs=[
            pl.BlockSpec((_BLK, D_lat), lambda i: (i, 0)),
            pl.BlockSpec((_BLK, D_in), lambda i: (i, 0)),
        ],
        out_shape=[
            jax.ShapeDtypeStruct((B, D_lat), jnp.float32),
            jax.ShapeDtypeStruct((B, D_in), jnp.float32),
        ],
    )(x, W_enc.astype(jnp.bfloat16), b_enc[None, :],
      W_dec.astype(jnp.bfloat16), b_dec[None, :])
    return (recon, sparse)


# speedup vs baseline: 1.1931x; 1.0103x over previous
import jax
import jax.numpy as jnp
from jax.experimental import pallas as pl

_K = 64
_BLK = 256


def _count(cond):
    c = cond.astype(jnp.int16)
    w = c.shape[1]
    while w > 128:
        w //= 2
        c = c[:, :w] + c[:, w:]
    return jnp.sum(c.astype(jnp.int32), axis=1, keepdims=True)


def _search16(plane, target, bits=15, signed=True):
    if signed:
        cnt0 = _count(plane >= jnp.int16(0))
        acc0 = cnt0 >= target
        base = jnp.where(acc0, jnp.int32(0), jnp.int32(-(2**15)))
        cnt_above = jnp.where(acc0, jnp.int32(0), cnt0)
    else:
        base = jnp.zeros_like(target, dtype=jnp.int32)
        cnt_above = jnp.zeros_like(target, dtype=jnp.int32)
    for b in range(bits - 1, -1, -1):
        trial = base | jnp.int32(1 << b)
        cnt = _count(plane >= trial.astype(jnp.int16))
        acc = cnt >= target
        base = jnp.where(acc, trial, base)
        cnt_above = jnp.where(acc, cnt_above, cnt)
    return base, cnt_above


_DN_T = (((1,), (1,)), ((), ()))


def _body(x_ref, we_ref, be_ref, wd_ref, bd_ref, sparse_ref, recon_ref):
    lat = jax.lax.dot_general(x_ref[...].astype(jnp.bfloat16), we_ref[...],
                              _DN_T, preferred_element_type=jnp.float32)
    lat = lat + be_ref[...]

    ik = jax.lax.bitcast_convert_type(lat, jnp.int32)
    keys = ik ^ ((ik >> 31) & jnp.int32(0x7FFFFFFF))
    hi = (keys >> 16).astype(jnp.int16)
    lo = ((keys & jnp.int32(0xFFFF)) - jnp.int32(32768)).astype(jnp.int16)

    h, c1 = _search16(hi, jnp.int32(_K))
    h16 = h.astype(jnp.int16)
    need = jnp.int32(_K) - c1
    lom = jnp.where(hi == h16, lo, jnp.int16(-(2**15)))
    l, c2 = _search16(lom, need)

    thr = (h << 16) | (l + 32768)
    above = keys > thr
    eq = keys == thr
    need_eq = jnp.int32(_K) - (c1 + c2)
    ncols = lat.shape[1]
    rev = jnp.int32(ncols - 1) - jax.lax.broadcasted_iota(jnp.int32, lat.shape, 1)
    ipm = jnp.where(eq, rev, jnp.int32(-(2**15))).astype(jnp.int16)
    d, _ = _search16(ipm, need_eq, bits=(ncols - 1).bit_length(),
                     signed=False)
    mask = above | (eq & (rev >= d))
    sparse = jnp.where(mask, lat, jnp.float32(0.0))
    sparse_ref[...] = sparse
    recon_ref[...] = jax.lax.dot_general(
        sparse.astype(jnp.bfloat16), wd_ref[...], _DN_T,
        preferred_element_type=jnp.float32) + bd_ref[...]


@jax.jit
def kernel(x, W_enc, b_enc, W_dec, b_dec):
    B, D_in = x.shape
    D_lat = W_enc.shape[0]
    grid = (B // _BLK,)
    sparse, recon = pl.pallas_call(
        _body,
        grid=grid,
        in_specs=[
            pl.BlockSpec((_BLK, D_in), lambda i: (i, 0)),
            pl.BlockSpec((D_lat, D_in), lambda i: (0, 0)),
            pl.BlockSpec((1, D_lat), lambda i: (0, 0)),
            pl.BlockSpec((D_in, D_lat), lambda i: (0, 0)),
            pl.BlockSpec((1, D_in), lambda i: (0, 0)),
        ],
        out_specs=[
            pl.BlockSpec((_BLK, D_lat), lambda i: (i, 0)),
            pl.BlockSpec((_BLK, D_in), lambda i: (i, 0)),
        ],
        out_shape=[
            jax.ShapeDtypeStruct((B, D_lat), jnp.float32),
            jax.ShapeDtypeStruct((B, D_in), jnp.float32),
        ],
    )(x, W_enc.astype(jnp.bfloat16), b_enc[None, :],
      W_dec.astype(jnp.bfloat16), b_dec[None, :])
    return (recon, sparse)
